# baseline (speedup 1.0000x reference)
import functools

import jax
import jax.numpy as jnp
from jax import lax
from jax.experimental import pallas as pl
from jax.experimental.pallas import tpu as pltpu
from jax.experimental.pallas import tpu_sc as plsc

_VOCAB = 1000
_EMBED = 64
_EPAD = 128
_BATCH = 16384

_NC = 2
_NS = 16
_NW = _NC * _NS
_BPW = _BATCH // _NW
_CHUNK = 128
_NCHUNK = _BPW // _CHUNK

_BBLK = 2048
_NBB = _BATCH // _BBLK


_sc_mesh = plsc.VectorSubcoreMesh(core_axis_name="c", subcore_axis_name="s")


@functools.partial(
    pl.kernel,
    mesh=_sc_mesh,
    out_type=jax.ShapeDtypeStruct((_BATCH, _EMBED), jnp.float32),
    scratch_types=[
        pltpu.VMEM((_BPW,), jnp.int32),
        pltpu.VMEM((_BPW, _EMBED), jnp.float32),
        pltpu.SemaphoreType.DMA,
        pltpu.SemaphoreType.DMA,
        pltpu.SemaphoreType.DMA,
        pltpu.SemaphoreType.DMA,
        pltpu.SemaphoreType.DMA,
    ],
    compiler_params=pltpu.CompilerParams(use_tc_tiling_on_sc=False),
)
def _sc_embed_gather(emb_hbm, idx_hbm, out_hbm, idx_v, rows_v, *sems):
    wid = lax.axis_index("s") * _NC + lax.axis_index("c")
    base = wid * _BPW
    pltpu.sync_copy(idx_hbm.at[pl.ds(base, _BPW)], idx_v)

    gsems = sems[:_NCHUNK]
    ssem = sems[_NCHUNK]

    gathers = [
        pltpu.async_copy(
            emb_hbm.at[idx_v.at[pl.ds(g * _CHUNK, _CHUNK)]],
            rows_v.at[pl.ds(g * _CHUNK, _CHUNK)],
            gsems[g],
        )
        for g in range(_NCHUNK)
    ]
    for g in gathers:
        g.wait()
    pltpu.async_copy(
        rows_v, out_hbm.at[pl.ds(base, _BPW)], ssem,
    ).wait()


def _proj_body(x_ref, w_ref, b_ref, o_ref):
    o_ref[...] = lax.dot_general(
        w_ref[...], x_ref[...],
        dimension_numbers=(((1,), (1,)), ((), ())),
        preferred_element_type=jnp.float32,
    ) + b_ref[...]


def _proj_matmul(X, W, b):
    return pl.pallas_call(
        _proj_body,
        grid=(_NBB,),
        in_specs=[
            pl.BlockSpec((_BBLK, _EMBED), lambda k: (k, 0)),
            pl.BlockSpec((_VOCAB, _EMBED), lambda k: (0, 0)),
            pl.BlockSpec((_VOCAB, 1), lambda k: (0, 0)),
        ],
        out_specs=pl.BlockSpec((_VOCAB, _BBLK), lambda k: (0, k)),
        out_shape=jax.ShapeDtypeStruct((_VOCAB, _BATCH), jnp.float32),
    )(X, W, b.reshape(_VOCAB, 1))


def kernel(target, emb_table, W, b):
    X = _sc_embed_gather(emb_table, target.astype(jnp.int32))
    outT = _proj_matmul(X, W, b)
    return outT.T


# speedup vs baseline: 1.0787x; 1.0787x over previous
import functools

import jax
import jax.numpy as jnp
from jax import lax
from jax.experimental import pallas as pl
from jax.experimental.pallas import tpu as pltpu
from jax.experimental.pallas import tpu_sc as plsc

_VOCAB = 1000
_EMBED = 64
_EPAD = 128
_BATCH = 16384

_NC = 2
_NS = 16
_NW = _NC * _NS
_BPW = _BATCH // _NW
_CHUNK = 128
_NCHUNK = _BPW // _CHUNK

_BBLK = 2048
_NBB = _BATCH // _BBLK


_sc_mesh = plsc.VectorSubcoreMesh(core_axis_name="c", subcore_axis_name="s")


@functools.partial(
    pl.kernel,
    mesh=_sc_mesh,
    out_type=jax.ShapeDtypeStruct((_BATCH, _EPAD), jnp.float32),
    scratch_types=[
        pltpu.VMEM((_BPW,), jnp.int32),
        pltpu.VMEM((_BPW, _EPAD), jnp.float32),
        pltpu.SemaphoreType.DMA,
        pltpu.SemaphoreType.DMA,
        pltpu.SemaphoreType.DMA,
        pltpu.SemaphoreType.DMA,
        pltpu.SemaphoreType.DMA,
    ],
)
def _sc_embed_gather(emb_hbm, idx_hbm, out_hbm, idx_v, rows_v, *sems):
    wid = lax.axis_index("s") * _NC + lax.axis_index("c")
    base = wid * _BPW
    pltpu.sync_copy(idx_hbm.at[pl.ds(base, _BPW)], idx_v)

    gsems = sems[:_NCHUNK]
    ssem = sems[_NCHUNK]

    gathers = [
        pltpu.async_copy(
            emb_hbm.at[idx_v.at[pl.ds(g * _CHUNK, _CHUNK)]],
            rows_v.at[pl.ds(g * _CHUNK, _CHUNK)],
            gsems[g],
        )
        for g in range(_NCHUNK)
    ]
    for g in gathers:
        g.wait()
    pltpu.async_copy(
        rows_v, out_hbm.at[pl.ds(base, _BPW)], ssem,
    ).wait()


def _proj_body(x_ref, w_ref, b_ref, o_ref):
    o_ref[...] = lax.dot_general(
        w_ref[...], x_ref[...],
        dimension_numbers=(((1,), (1,)), ((), ())),
        preferred_element_type=jnp.float32,
    ) + b_ref[...]


def _proj_matmul(X, W, b):
    w_pad = jnp.pad(W, ((0, 0), (0, _EPAD - _EMBED)))
    return pl.pallas_call(
        _proj_body,
        grid=(_NBB,),
        in_specs=[
            pl.BlockSpec((_BBLK, _EPAD), lambda k: (k, 0)),
            pl.BlockSpec((_VOCAB, _EPAD), lambda k: (0, 0)),
            pl.BlockSpec((_VOCAB, 1), lambda k: (0, 0)),
        ],
        out_specs=pl.BlockSpec((_VOCAB, _BBLK), lambda k: (0, k)),
        out_shape=jax.ShapeDtypeStruct((_VOCAB, _BATCH), jnp.float32),
    )(X, w_pad, b.reshape(_VOCAB, 1))


def kernel(target, emb_table, W, b):
    emb_pad = jnp.pad(emb_table, ((0, 0), (0, _EPAD - _EMBED)))
    X = _sc_embed_gather(emb_pad, target.astype(jnp.int32))
    outT = _proj_matmul(X, W, b)
    return outT.T
